# Initial kernel scaffold; baseline (speedup 1.0000x reference)
import jax
import jax.numpy as jnp
from jax.experimental import pallas as pl


def kernel(x, edge_index, W0, b0, g0, be0, W1, b1, g1, be1, aW1, ab1, aW2, ab2):
    raise NotImplementedError("write your pallas kernel here")


# speedup vs baseline: 12.3145x; 12.3145x over previous
import functools

import jax
import jax.numpy as jnp
from jax import lax
from jax.experimental import pallas as pl
from jax.experimental.pallas import tpu as pltpu
from jax.experimental.pallas import tpu_sc as plsc

N = 10000
E = 320000
D = 128
HID = 128
HH = HID // 2
K = 16

NC = 2
NS = 16
CHUNK = 128
NCHUNK = 160
EDGES_PER_TILE = CHUNK * NCHUNK
E_PAD = NS * EDGES_PER_TILE
NCHUNK_D = NCHUNK // NC
ROWS_PAD = 10112
ROWS_PER_TILE = ROWS_PAD // NS
DUMMY = N

RB = 1000
NB = N // RB

_MESH = plsc.VectorSubcoreMesh(core_axis_name="c", subcore_axis_name="s")


@functools.partial(
    pl.kernel,
    out_type=jax.ShapeDtypeStruct((NC, ROWS_PAD, 8), jnp.float32),
    mesh=_MESH,
    compiler_params=pltpu.CompilerParams(use_tc_tiling_on_sc=False),
    scratch_types=[
        pltpu.VMEM((NCHUNK_D, CHUNK), jnp.int32),
        pltpu.VMEM((CHUNK, 8), jnp.float32),
        pltpu.VMEM_SHARED((ROWS_PAD, 8), jnp.float32),
    ],
)
def _sc_degree(dst_hbm, ones_hbm, zeros_hbm, out_hbm, dst_v, ones_v, deg_sh):
    c = lax.axis_index("c")
    s = lax.axis_index("s")
    r0 = s * ROWS_PER_TILE
    pltpu.sync_copy(zeros_hbm.at[pl.ds(r0, ROWS_PER_TILE)],
                    deg_sh.at[pl.ds(r0, ROWS_PER_TILE)])
    pltpu.sync_copy(dst_hbm.at[s].at[pl.ds(c * NCHUNK_D, NCHUNK_D)], dst_v)
    pltpu.sync_copy(ones_hbm, ones_v)
    plsc.subcore_barrier()

    def body(j, carry):
        pltpu.sync_copy(ones_v, deg_sh.at[dst_v.at[j]], add=True)
        return carry

    lax.fori_loop(0, NCHUNK_D, body, 0)
    plsc.subcore_barrier()
    pltpu.sync_copy(deg_sh.at[pl.ds(r0, ROWS_PER_TILE)],
                    out_hbm.at[c].at[pl.ds(r0, ROWS_PER_TILE)])


@functools.partial(
    pl.kernel,
    out_type=jax.ShapeDtypeStruct((NC, ROWS_PAD, HH), jnp.float32),
    mesh=_MESH,
    compiler_params=pltpu.CompilerParams(use_tc_tiling_on_sc=False),
    scratch_types=[
        pltpu.VMEM((NCHUNK, CHUNK), jnp.int32),
        pltpu.VMEM((NCHUNK, CHUNK), jnp.int32),
        pltpu.VMEM((CHUNK, HH), jnp.float32),
        pltpu.VMEM((CHUNK, HH), jnp.float32),
        pltpu.VMEM_SHARED((ROWS_PAD, HH), jnp.float32),
        pltpu.SemaphoreType.DMA,
        pltpu.SemaphoreType.DMA,
    ],
)
def _sc_agg(y_hbm, src_hbm, dst_hbm, zeros_hbm, out_hbm,
            src_v, dst_v, buf0, buf1, agg_sh, sem0, sem1):
    c = lax.axis_index("c")
    s = lax.axis_index("s")
    r0 = s * ROWS_PER_TILE
    pltpu.sync_copy(zeros_hbm.at[pl.ds(r0, ROWS_PER_TILE)],
                    agg_sh.at[pl.ds(r0, ROWS_PER_TILE)])
    pltpu.sync_copy(src_hbm.at[s], src_v)
    pltpu.sync_copy(dst_hbm.at[s], dst_v)
    plsc.subcore_barrier()

    ytab = y_hbm.at[c]
    pltpu.async_copy(ytab.at[src_v.at[0]], buf0, sem0)

    def body(t, carry):
        j0 = 2 * t
        pltpu.async_copy(ytab.at[src_v.at[j0 + 1]], buf1, sem1)
        pltpu.make_async_copy(ytab.at[src_v.at[j0]], buf0, sem0).wait()
        pltpu.sync_copy(buf0, agg_sh.at[dst_v.at[j0]], add=True)

        @pl.when(j0 + 2 < NCHUNK)
        def _():
            pltpu.async_copy(ytab.at[src_v.at[j0 + 2]], buf0, sem0)

        pltpu.make_async_copy(ytab.at[src_v.at[j0 + 1]], buf1, sem1).wait()
        pltpu.sync_copy(buf1, agg_sh.at[dst_v.at[j0 + 1]], add=True)
        return carry

    lax.fori_loop(0, NCHUNK // 2, body, 0)
    plsc.subcore_barrier()
    pltpu.sync_copy(agg_sh.at[pl.ds(r0, ROWS_PER_TILE)],
                    out_hbm.at[c].at[pl.ds(r0, ROWS_PER_TILE)])


def _tc_pre_body(x_ref, w_ref, deg_ref, y_ref, dinv_ref):
    deg = deg_ref[0, :, 0:1] + deg_ref[1, :, 0:1] + 1.0
    dinv = lax.rsqrt(deg)
    xw = jnp.dot(x_ref[...], w_ref[...], preferred_element_type=jnp.float32)
    y = xw * dinv
    y_ref[0] = y[:, :HH]
    y_ref[1] = y[:, HH:]
    dinv_ref[...] = dinv


def _tc_pre(x, w0, deg2):
    return pl.pallas_call(
        _tc_pre_body,
        grid=(NB,),
        in_specs=[
            pl.BlockSpec((RB, D), lambda i: (i, 0)),
            pl.BlockSpec((D, HID), lambda i: (0, 0)),
            pl.BlockSpec((NC, RB, 8), lambda i: (0, i, 0)),
        ],
        out_specs=[
            pl.BlockSpec((NC, RB, HH), lambda i: (0, i, 0)),
            pl.BlockSpec((RB, 1), lambda i: (i, 0)),
        ],
        out_shape=[
            jax.ShapeDtypeStruct((NC, N, HH), jnp.float32),
            jax.ShapeDtypeStruct((N, 1), jnp.float32),
        ],
    )(x, w0, deg2)


def _z_of(agg_ref, y_ref, dinv_ref, b_ref):
    agg = jnp.concatenate([agg_ref[0], agg_ref[1]], axis=1)
    y = jnp.concatenate([y_ref[0], y_ref[1]], axis=1)
    return (agg + y) * dinv_ref[...] + b_ref[...]


def _bn_relu(z, ssum, ssq, g_ref, be_ref):
    mu = ssum[...] * (1.0 / N)
    var = ssq[...] * (1.0 / N) - mu * mu
    return jnp.maximum((z - mu) * lax.rsqrt(var + 1e-5) * g_ref[...]
                       + be_ref[...], 0.0)


def _tc_mid1_body(agg_ref, y_ref, dinv_ref, b_ref, g_ref, be_ref, w_ref,
                  out_ref, ssum, ssq):
    p = pl.program_id(0)
    i = pl.program_id(1)
    z = _z_of(agg_ref, y_ref, dinv_ref, b_ref)

    @pl.when(jnp.logical_and(p == 0, i == 0))
    def _():
        ssum[...] = jnp.zeros_like(ssum)
        ssq[...] = jnp.zeros_like(ssq)

    @pl.when(p == 0)
    def _():
        ssum[...] += jnp.sum(z, axis=0, keepdims=True)
        ssq[...] += jnp.sum(z * z, axis=0, keepdims=True)

    @pl.when(p == 1)
    def _():
        h = _bn_relu(z, ssum, ssq, g_ref, be_ref)
        y1 = jnp.dot(h, w_ref[...],
                     preferred_element_type=jnp.float32) * dinv_ref[...]
        out_ref[0] = y1[:, :HH]
        out_ref[1] = y1[:, HH:]


def _tc_mid1(agg2, y0, dinv, b, g, be, w1):
    return pl.pallas_call(
        _tc_mid1_body,
        grid=(2, NB),
        in_specs=[
            pl.BlockSpec((NC, RB, HH), lambda p, i: (0, i, 0)),
            pl.BlockSpec((NC, RB, HH), lambda p, i: (0, i, 0)),
            pl.BlockSpec((RB, 1), lambda p, i: (i, 0)),
            pl.BlockSpec((1, HID), lambda p, i: (0, 0)),
            pl.BlockSpec((1, HID), lambda p, i: (0, 0)),
            pl.BlockSpec((1, HID), lambda p, i: (0, 0)),
            pl.BlockSpec((HID, HID), lambda p, i: (0, 0)),
        ],
        out_specs=pl.BlockSpec((NC, RB, HH), lambda p, i: (0, i, 0)),
        out_shape=jax.ShapeDtypeStruct((NC, N, HH), jnp.float32),
        scratch_shapes=[
            pltpu.VMEM((1, HID), jnp.float32),
            pltpu.VMEM((1, HID), jnp.float32),
        ],
    )(agg2, y0, dinv, b, g, be, w1)


def _tc_mid2_body(agg_ref, y_ref, dinv_ref, b_ref, g_ref, be_ref,
                  aw1_ref, ab1_ref, aw2_ref, ab2_ref,
                  h_ref, s_ref, ssum, ssq):
    p = pl.program_id(0)
    i = pl.program_id(1)
    z = _z_of(agg_ref, y_ref, dinv_ref, b_ref)

    @pl.when(jnp.logical_and(p == 0, i == 0))
    def _():
        ssum[...] = jnp.zeros_like(ssum)
        ssq[...] = jnp.zeros_like(ssq)

    @pl.when(p == 0)
    def _():
        ssum[...] += jnp.sum(z, axis=0, keepdims=True)
        ssq[...] += jnp.sum(z * z, axis=0, keepdims=True)

    @pl.when(p == 1)
    def _():
        h = _bn_relu(z, ssum, ssq, g_ref, be_ref)
        h_ref[...] = h
        hid = jnp.maximum(
            jnp.dot(h, aw1_ref[...], preferred_element_type=jnp.float32)
            + ab1_ref[...], 0.0)
        logits = jnp.dot(hid, aw2_ref[...],
                         preferred_element_type=jnp.float32) + ab2_ref[...]
        m = jnp.max(logits, axis=-1, keepdims=True)
        e = jnp.exp(logits - m)
        s_ref[...] = e / jnp.sum(e, axis=-1, keepdims=True)


def _tc_mid2(agg2, y1, dinv, b, g, be, aw1, ab1, aw2, ab2):
    return pl.pallas_call(
        _tc_mid2_body,
        grid=(2, NB),
        in_specs=[
            pl.BlockSpec((NC, RB, HH), lambda p, i: (0, i, 0)),
            pl.BlockSpec((NC, RB, HH), lambda p, i: (0, i, 0)),
            pl.BlockSpec((RB, 1), lambda p, i: (i, 0)),
            pl.BlockSpec((1, HID), lambda p, i: (0, 0)),
            pl.BlockSpec((1, HID), lambda p, i: (0, 0)),
            pl.BlockSpec((1, HID), lambda p, i: (0, 0)),
            pl.BlockSpec((HID, HID), lambda p, i: (0, 0)),
            pl.BlockSpec((1, HID), lambda p, i: (0, 0)),
            pl.BlockSpec((HID, K), lambda p, i: (0, 0)),
            pl.BlockSpec((1, K), lambda p, i: (0, 0)),
        ],
        out_specs=[
            pl.BlockSpec((RB, HID), lambda p, i: (i, 0)),
            pl.BlockSpec((RB, K), lambda p, i: (i, 0)),
        ],
        out_shape=[
            jax.ShapeDtypeStruct((N, HID), jnp.float32),
            jax.ShapeDtypeStruct((N, K), jnp.float32),
        ],
        scratch_shapes=[
            pltpu.VMEM((1, HID), jnp.float32),
            pltpu.VMEM((1, HID), jnp.float32),
        ],
    )(agg2, y1, dinv, b, g, be, aw1, ab1, aw2, ab2)


def kernel(x, edge_index, W0, b0, g0, be0, W1, b1, g1, be1, aW1, ab1, aW2, ab2):
    pad = E_PAD - E
    src = jnp.concatenate(
        [edge_index[0], jnp.zeros((pad,), jnp.int32)]).reshape(NS, NCHUNK, CHUNK)
    dst = jnp.concatenate(
        [edge_index[1], jnp.full((pad,), DUMMY, jnp.int32)]).reshape(NS, NCHUNK, CHUNK)
    zeros8 = jnp.zeros((ROWS_PAD, 8), jnp.float32)
    ones8 = jnp.ones((CHUNK, 8), jnp.float32)
    zerosH = jnp.zeros((ROWS_PAD, HH), jnp.float32)

    deg2 = _sc_degree(dst, ones8, zeros8)
    y0, dinv = _tc_pre(x, W0, deg2)
    agg0 = _sc_agg(y0, src, dst, zerosH)
    y1 = _tc_mid1(agg0, y0, dinv, b0.reshape(1, -1), g0.reshape(1, -1),
                  be0.reshape(1, -1), W1)
    agg1 = _sc_agg(y1, src, dst, zerosH)
    h, S = _tc_mid2(agg1, y1, dinv, b1.reshape(1, -1), g1.reshape(1, -1),
                    be1.reshape(1, -1), aW1, ab1.reshape(1, -1), aW2,
                    ab2.reshape(1, -1))
    return (h, S)


# speedup vs baseline: 13.0137x; 1.0568x over previous
import functools

import jax
import jax.numpy as jnp
from jax import lax
from jax.experimental import pallas as pl
from jax.experimental.pallas import tpu as pltpu
from jax.experimental.pallas import tpu_sc as plsc

N = 10000
E = 320000
D = 128
HID = 128
HH = HID // 2
K = 16

NC = 2
NS = 16
CHUNK = 128
NCHUNK = 160
EDGES_PER_TILE = CHUNK * NCHUNK
E_PAD = NS * EDGES_PER_TILE
NCHUNK_D = NCHUNK // NC
ROWS_PAD = 10112
ROWS_PER_TILE = ROWS_PAD // NS
DUMMY = N

RB = 1000
NB = N // RB

_MESH = plsc.VectorSubcoreMesh(core_axis_name="c", subcore_axis_name="s")


@functools.partial(
    pl.kernel,
    out_type=jax.ShapeDtypeStruct((NC, ROWS_PAD, 8), jnp.float32),
    mesh=_MESH,
    compiler_params=pltpu.CompilerParams(use_tc_tiling_on_sc=False),
    scratch_types=[
        pltpu.VMEM((NCHUNK_D, CHUNK), jnp.int32),
        pltpu.VMEM((CHUNK, 8), jnp.float32),
        pltpu.VMEM_SHARED((ROWS_PAD, 8), jnp.float32),
    ],
)
def _sc_degree(dst_hbm, ones_hbm, zeros_hbm, out_hbm, dst_v, ones_v, deg_sh):
    c = lax.axis_index("c")
    s = lax.axis_index("s")
    r0 = s * ROWS_PER_TILE
    pltpu.sync_copy(zeros_hbm.at[pl.ds(r0, ROWS_PER_TILE)],
                    deg_sh.at[pl.ds(r0, ROWS_PER_TILE)])
    pltpu.sync_copy(dst_hbm.at[s].at[pl.ds(c * NCHUNK_D, NCHUNK_D)], dst_v)
    pltpu.sync_copy(ones_hbm, ones_v)
    plsc.subcore_barrier()

    def body(j, carry):
        pltpu.sync_copy(ones_v, deg_sh.at[dst_v.at[j]], add=True)
        return carry

    lax.fori_loop(0, NCHUNK_D, body, 0)
    plsc.subcore_barrier()
    pltpu.sync_copy(deg_sh.at[pl.ds(r0, ROWS_PER_TILE)],
                    out_hbm.at[c].at[pl.ds(r0, ROWS_PER_TILE)])


GRP = 2
NGRP = NCHUNK // GRP


@functools.partial(
    pl.kernel,
    out_type=jax.ShapeDtypeStruct((NC, ROWS_PAD, HH), jnp.float32),
    mesh=_MESH,
    compiler_params=pltpu.CompilerParams(use_tc_tiling_on_sc=False),
    scratch_types=[
        pltpu.VMEM((NCHUNK, CHUNK), jnp.int32),
        pltpu.VMEM((NCHUNK, CHUNK), jnp.int32),
        pltpu.VMEM((2 * GRP, CHUNK, HH), jnp.float32),
        pltpu.VMEM_SHARED((ROWS_PAD, HH), jnp.float32),
        [pltpu.SemaphoreType.DMA] * (2 * GRP),
        [pltpu.SemaphoreType.DMA] * (2 * GRP),
    ],
)
def _sc_agg(y_hbm, src_hbm, dst_hbm, zeros_hbm, out_hbm,
            src_v, dst_v, bufs, agg_sh, gsems, ssems):
    c = lax.axis_index("c")
    s = lax.axis_index("s")
    r0 = s * ROWS_PER_TILE
    pltpu.sync_copy(zeros_hbm.at[pl.ds(r0, ROWS_PER_TILE)],
                    agg_sh.at[pl.ds(r0, ROWS_PER_TILE)])
    pltpu.sync_copy(src_hbm.at[s], src_v)
    pltpu.sync_copy(dst_hbm.at[s], dst_v)
    plsc.subcore_barrier()

    ytab = y_hbm.at[c]

    def gather(j, b):
        pltpu.async_copy(ytab.at[src_v.at[j]], bufs.at[b], gsems[b])

    for b in range(2 * GRP):
        gather(b, b)

    def group_body(g, half):
        for k in range(GRP):
            b = half * GRP + k
            j = g * GRP + k
            pltpu.make_async_copy(ytab.at[src_v.at[j]], bufs.at[b],
                                  gsems[b]).wait()
            pltpu.async_copy(bufs.at[b], agg_sh.at[dst_v.at[j]], ssems[b],
                             add=True)
        for k in range(GRP):
            b = half * GRP + k
            j = g * GRP + k
            pltpu.make_async_copy(bufs.at[b], agg_sh.at[dst_v.at[j]],
                                  ssems[b]).wait()

        @pl.when(g + 2 < NGRP)
        def _():
            for k in range(GRP):
                gather((g + 2) * GRP + k, half * GRP + k)

    def body(t, carry):
        group_body(2 * t, 0)
        group_body(2 * t + 1, 1)
        return carry

    lax.fori_loop(0, NGRP // 2, body, 0)
    plsc.subcore_barrier()
    pltpu.sync_copy(agg_sh.at[pl.ds(r0, ROWS_PER_TILE)],
                    out_hbm.at[c].at[pl.ds(r0, ROWS_PER_TILE)])


def _tc_pre_body(x_ref, w_ref, deg_ref, y_ref, dinv_ref):
    deg = deg_ref[0, :, 0:1] + deg_ref[1, :, 0:1] + 1.0
    dinv = lax.rsqrt(deg)
    xw = jnp.dot(x_ref[...], w_ref[...], preferred_element_type=jnp.float32)
    y = xw * dinv
    y_ref[0] = y[:, :HH]
    y_ref[1] = y[:, HH:]
    dinv_ref[...] = dinv


def _tc_pre(x, w0, deg2):
    return pl.pallas_call(
        _tc_pre_body,
        grid=(NB,),
        in_specs=[
            pl.BlockSpec((RB, D), lambda i: (i, 0)),
            pl.BlockSpec((D, HID), lambda i: (0, 0)),
            pl.BlockSpec((NC, RB, 8), lambda i: (0, i, 0)),
        ],
        out_specs=[
            pl.BlockSpec((NC, RB, HH), lambda i: (0, i, 0)),
            pl.BlockSpec((RB, 1), lambda i: (i, 0)),
        ],
        out_shape=[
            jax.ShapeDtypeStruct((NC, N, HH), jnp.float32),
            jax.ShapeDtypeStruct((N, 1), jnp.float32),
        ],
    )(x, w0, deg2)


def _z_of(agg_ref, y_ref, dinv_ref, b_ref):
    agg = jnp.concatenate([agg_ref[0], agg_ref[1]], axis=1)
    y = jnp.concatenate([y_ref[0], y_ref[1]], axis=1)
    return (agg + y) * dinv_ref[...] + b_ref[...]


def _bn_relu(z, ssum, ssq, g_ref, be_ref):
    mu = ssum[...] * (1.0 / N)
    var = ssq[...] * (1.0 / N) - mu * mu
    return jnp.maximum((z - mu) * lax.rsqrt(var + 1e-5) * g_ref[...]
                       + be_ref[...], 0.0)


def _tc_mid1_body(agg_ref, y_ref, dinv_ref, b_ref, g_ref, be_ref, w_ref,
                  out_ref, ssum, ssq):
    p = pl.program_id(0)
    i = pl.program_id(1)
    z = _z_of(agg_ref, y_ref, dinv_ref, b_ref)

    @pl.when(jnp.logical_and(p == 0, i == 0))
    def _():
        ssum[...] = jnp.zeros_like(ssum)
        ssq[...] = jnp.zeros_like(ssq)

    @pl.when(p == 0)
    def _():
        ssum[...] += jnp.sum(z, axis=0, keepdims=True)
        ssq[...] += jnp.sum(z * z, axis=0, keepdims=True)

    @pl.when(p == 1)
    def _():
        h = _bn_relu(z, ssum, ssq, g_ref, be_ref)
        y1 = jnp.dot(h, w_ref[...],
                     preferred_element_type=jnp.float32) * dinv_ref[...]
        out_ref[0] = y1[:, :HH]
        out_ref[1] = y1[:, HH:]


def _tc_mid1(agg2, y0, dinv, b, g, be, w1):
    return pl.pallas_call(
        _tc_mid1_body,
        grid=(2, NB),
        in_specs=[
            pl.BlockSpec((NC, RB, HH), lambda p, i: (0, i, 0)),
            pl.BlockSpec((NC, RB, HH), lambda p, i: (0, i, 0)),
            pl.BlockSpec((RB, 1), lambda p, i: (i, 0)),
            pl.BlockSpec((1, HID), lambda p, i: (0, 0)),
            pl.BlockSpec((1, HID), lambda p, i: (0, 0)),
            pl.BlockSpec((1, HID), lambda p, i: (0, 0)),
            pl.BlockSpec((HID, HID), lambda p, i: (0, 0)),
        ],
        out_specs=pl.BlockSpec((NC, RB, HH), lambda p, i: (0, i, 0)),
        out_shape=jax.ShapeDtypeStruct((NC, N, HH), jnp.float32),
        scratch_shapes=[
            pltpu.VMEM((1, HID), jnp.float32),
            pltpu.VMEM((1, HID), jnp.float32),
        ],
    )(agg2, y0, dinv, b, g, be, w1)


def _tc_mid2_body(agg_ref, y_ref, dinv_ref, b_ref, g_ref, be_ref,
                  aw1_ref, ab1_ref, aw2_ref, ab2_ref,
                  h_ref, s_ref, ssum, ssq):
    p = pl.program_id(0)
    i = pl.program_id(1)
    z = _z_of(agg_ref, y_ref, dinv_ref, b_ref)

    @pl.when(jnp.logical_and(p == 0, i == 0))
    def _():
        ssum[...] = jnp.zeros_like(ssum)
        ssq[...] = jnp.zeros_like(ssq)

    @pl.when(p == 0)
    def _():
        ssum[...] += jnp.sum(z, axis=0, keepdims=True)
        ssq[...] += jnp.sum(z * z, axis=0, keepdims=True)

    @pl.when(p == 1)
    def _():
        h = _bn_relu(z, ssum, ssq, g_ref, be_ref)
        h_ref[...] = h
        hid = jnp.maximum(
            jnp.dot(h, aw1_ref[...], preferred_element_type=jnp.float32)
            + ab1_ref[...], 0.0)
        logits = jnp.dot(hid, aw2_ref[...],
                         preferred_element_type=jnp.float32) + ab2_ref[...]
        m = jnp.max(logits, axis=-1, keepdims=True)
        e = jnp.exp(logits - m)
        s_ref[...] = e / jnp.sum(e, axis=-1, keepdims=True)


def _tc_mid2(agg2, y1, dinv, b, g, be, aw1, ab1, aw2, ab2):
    return pl.pallas_call(
        _tc_mid2_body,
        grid=(2, NB),
        in_specs=[
            pl.BlockSpec((NC, RB, HH), lambda p, i: (0, i, 0)),
            pl.BlockSpec((NC, RB, HH), lambda p, i: (0, i, 0)),
            pl.BlockSpec((RB, 1), lambda p, i: (i, 0)),
            pl.BlockSpec((1, HID), lambda p, i: (0, 0)),
            pl.BlockSpec((1, HID), lambda p, i: (0, 0)),
            pl.BlockSpec((1, HID), lambda p, i: (0, 0)),
            pl.BlockSpec((HID, HID), lambda p, i: (0, 0)),
            pl.BlockSpec((1, HID), lambda p, i: (0, 0)),
            pl.BlockSpec((HID, K), lambda p, i: (0, 0)),
            pl.BlockSpec((1, K), lambda p, i: (0, 0)),
        ],
        out_specs=[
            pl.BlockSpec((RB, HID), lambda p, i: (i, 0)),
            pl.BlockSpec((RB, K), lambda p, i: (i, 0)),
        ],
        out_shape=[
            jax.ShapeDtypeStruct((N, HID), jnp.float32),
            jax.ShapeDtypeStruct((N, K), jnp.float32),
        ],
        scratch_shapes=[
            pltpu.VMEM((1, HID), jnp.float32),
            pltpu.VMEM((1, HID), jnp.float32),
        ],
    )(agg2, y1, dinv, b, g, be, aw1, ab1, aw2, ab2)


def kernel(x, edge_index, W0, b0, g0, be0, W1, b1, g1, be1, aW1, ab1, aW2, ab2):
    pad = E_PAD - E
    src = jnp.concatenate(
        [edge_index[0], jnp.zeros((pad,), jnp.int32)]).reshape(NS, NCHUNK, CHUNK)
    dst = jnp.concatenate(
        [edge_index[1], jnp.full((pad,), DUMMY, jnp.int32)]).reshape(NS, NCHUNK, CHUNK)
    zeros8 = jnp.zeros((ROWS_PAD, 8), jnp.float32)
    ones8 = jnp.ones((CHUNK, 8), jnp.float32)
    zerosH = jnp.zeros((ROWS_PAD, HH), jnp.float32)

    deg2 = _sc_degree(dst, ones8, zeros8)
    y0, dinv = _tc_pre(x, W0, deg2)
    agg0 = _sc_agg(y0, src, dst, zerosH)
    y1 = _tc_mid1(agg0, y0, dinv, b0.reshape(1, -1), g0.reshape(1, -1),
                  be0.reshape(1, -1), W1)
    agg1 = _sc_agg(y1, src, dst, zerosH)
    h, S = _tc_mid2(agg1, y1, dinv, b1.reshape(1, -1), g1.reshape(1, -1),
                    be1.reshape(1, -1), aW1, ab1.reshape(1, -1), aW2,
                    ab2.reshape(1, -1))
    return (h, S)
